# Initial kernel scaffold; baseline (speedup 1.0000x reference)
import jax
import jax.numpy as jnp
from jax.experimental import pallas as pl


def kernel(latent, Wp, bp, pos_embed):
    raise NotImplementedError("write your pallas kernel here")


# speedup vs baseline: 2.6734x; 2.6734x over previous
import jax
import jax.numpy as jnp
from jax.experimental import pallas as pl
from jax.experimental.pallas import tpu as pltpu

_PATCH = 2
_EMBED = 768
_MAX_TOK = 2048


def _body(x_ref, w_ref, bpos_ref, o_ref):
    x = x_ref[0, 0]
    w = w_ref[...]
    acc = jnp.dot(x, w, preferred_element_type=jnp.float32)
    o_ref[0, 0] = acc + bpos_ref[...]


def kernel(latent, Wp, bp, pos_embed):
    B, C, F, H, W = latent.shape
    ph, pw = H // _PATCH, W // _PATCH
    T = ph * pw
    gsz = _MAX_TOK // T
    ng = B // gsz
    K = C * _PATCH * _PATCH

    x = latent.reshape(B, C, F, ph, _PATCH, pw, _PATCH)
    x = x.transpose(0, 2, 3, 5, 1, 4, 6).reshape(B, F, T, K)
    w = Wp.reshape(_EMBED, K).T
    bpos = pos_embed + bp[None, :]

    grid = (ng, F, gsz)
    out = pl.pallas_call(
        _body,
        grid=grid,
        in_specs=[
            pl.BlockSpec((1, 1, T, K), lambda g, f, v: (gsz * g + v, f, 0, 0)),
            pl.BlockSpec((K, _EMBED), lambda g, f, v: (0, 0)),
            pl.BlockSpec((T, _EMBED), lambda g, f, v: (0, 0)),
        ],
        out_specs=pl.BlockSpec((1, 1, T, _EMBED), lambda g, f, v: (g, f, v, 0)),
        out_shape=jax.ShapeDtypeStruct((ng, F, _MAX_TOK, _EMBED), jnp.float32),
        compiler_params=pltpu.CompilerParams(
            dimension_semantics=("parallel", "parallel", "parallel"),
        ),
    )(x, w, bpos)

    batched_idx = jnp.tile(
        jnp.repeat(jnp.arange(gsz, dtype=jnp.int32), T), (ng, 1)
    )
    return (out, batched_idx)


# speedup vs baseline: 2.8989x; 1.0844x over previous
import jax
import jax.numpy as jnp
from jax.experimental import pallas as pl
from jax.experimental.pallas import tpu as pltpu

_PATCH = 2
_EMBED = 768
_MAX_TOK = 2048


def _body(x_ref, w_ref, bpos_ref, o_ref):
    F = x_ref.shape[1]
    w = w_ref[...]
    for f in range(F):
        x = x_ref[0, f]
        acc = jnp.dot(x, w, preferred_element_type=jnp.float32)
        o_ref[0, f] = acc + bpos_ref[...]


def kernel(latent, Wp, bp, pos_embed):
    B, C, F, H, W = latent.shape
    ph, pw = H // _PATCH, W // _PATCH
    T = ph * pw
    gsz = _MAX_TOK // T
    ng = B // gsz
    K = C * _PATCH * _PATCH

    x = latent.reshape(B, C, F, ph, _PATCH, pw, _PATCH)
    x = x.transpose(0, 2, 3, 5, 1, 4, 6).reshape(B, F, T, K)
    w = Wp.reshape(_EMBED, K).T
    bpos = pos_embed + bp[None, :]

    grid = (ng, gsz)
    out = pl.pallas_call(
        _body,
        grid=grid,
        in_specs=[
            pl.BlockSpec((1, F, T, K), lambda g, v: (gsz * g + v, 0, 0, 0)),
            pl.BlockSpec((K, _EMBED), lambda g, v: (0, 0)),
            pl.BlockSpec((T, _EMBED), lambda g, v: (0, 0)),
        ],
        out_specs=pl.BlockSpec((1, F, T, _EMBED), lambda g, v: (g, 0, v, 0)),
        out_shape=jax.ShapeDtypeStruct((ng, F, _MAX_TOK, _EMBED), jnp.float32),
        compiler_params=pltpu.CompilerParams(
            dimension_semantics=("parallel", "parallel"),
        ),
    )(x, w, bpos)

    batched_idx = jnp.tile(
        jnp.repeat(jnp.arange(gsz, dtype=jnp.int32), T), (ng, 1)
    )
    return (out, batched_idx)
